# baseline (speedup 1.0000x reference)
import functools

import jax
import jax.numpy as jnp
from jax import lax
from jax.experimental import pallas as pl
from jax.experimental.pallas import tpu as pltpu
from jax.experimental.pallas import tpu_sc as plsc

VOCAB = 1000000
EMBED = 64
HIDDEN = 256
CLASSES = 5
BATCH = 4096
SEQ = 200

_INFO = plsc.get_sparse_core_info()
NC = _INFO.num_cores
NS = _INFO.num_subcores
LANES = _INFO.num_lanes
NW = NC * NS
BPW = BATCH // NW
C0 = 104
C1 = SEQ - C0
EG = EMBED // LANES
HALF = SEQ // 2
PAIR = 2 * EMBED
CB = 16384


def _pair_body(tt_ref, o_ref):
    xt = tt_ref[...].T.reshape(CB // 2, 2, EMBED)
    o_ref[...] = jnp.concatenate([xt[:, 0, :], xt[:, 1, :]], axis=1)


def _pair_table(tt):
    nb = (VOCAB + CB - 1) // CB
    return pl.pallas_call(
        _pair_body,
        grid=(nb,),
        in_specs=[pl.BlockSpec((EMBED, CB), lambda j: (0, j))],
        out_specs=pl.BlockSpec((CB // 2, PAIR), lambda j: (j, 0)),
        out_shape=jax.ShapeDtypeStruct((VOCAB // 2, PAIR), jnp.float32),
    )(tt)


def _pool_body(idx_hbm, par_hbm, tp_hbm, out_hbm, idx_v, rows_a, rows_b,
               par_a, par_b, cmp_v, out_v, sem_a, sem_b):
    wid = lax.axis_index("s") * NC + lax.axis_index("c")
    base = pl.multiple_of(wid * BPW, BPW)
    pltpu.sync_copy(idx_hbm.at[pl.ds(base * SEQ, BPW * SEQ)], idx_v)

    def start(e, buf, parbuf, sem):
        off = pl.multiple_of(e * SEQ, 8)
        pltpu.make_async_copy(
            par_hbm.at[pl.ds(base * SEQ + off, SEQ)],
            parbuf.at[pl.ds(0, SEQ)], sem).start()
        pltpu.make_async_copy(
            tp_hbm.at[idx_v.at[pl.ds(off, C0)]],
            buf.at[pl.ds(0, C0)], sem).start()
        pltpu.make_async_copy(
            tp_hbm.at[idx_v.at[pl.ds(off + C0, C1)]],
            buf.at[pl.ds(C0, C1)], sem).start()

    def wait(buf, parbuf, sem):
        pltpu.make_async_copy(
            par_hbm.at[pl.ds(base * SEQ, SEQ)],
            parbuf.at[pl.ds(0, SEQ)], sem).wait()
        pltpu.make_async_copy(
            tp_hbm.at[idx_v.at[pl.ds(0, C0)]],
            buf.at[pl.ds(0, C0)], sem).wait()
        pltpu.make_async_copy(
            tp_hbm.at[idx_v.at[pl.ds(C0, C1)]],
            buf.at[pl.ds(C0, C1)], sem).wait()

    def reduce(buf, parbuf, i):
        lanes = lax.iota(jnp.int32, LANES)

        def compact_group(g, _):
            rows = g * LANES + lanes
            msk = rows < SEQ
            parvec = parbuf[pl.ds(pl.multiple_of(g * LANES, 8), LANES)]
            for c in range(EMBED):
                val = plsc.load_gather(buf, [rows, parvec + c], mask=msk)
                plsc.store_scatter(cmp_v, [rows, lanes * 0 + c], val,
                                   mask=msk)
            return 0

        lax.fori_loop(0, (SEQ + LANES - 1) // LANES, compact_group, 0)

        def body(s, accs):
            new = []
            for j in range(2):
                for k in range(EG):
                    new.append(accs[j * EG + k]
                               + cmp_v[j * HALF + s, pl.ds(k * LANES, LANES)])
            return tuple(new)

        zero = jnp.zeros((LANES,), jnp.float32)
        accs = lax.fori_loop(0, HALF, body, (zero,) * (2 * EG))
        for k in range(EG):
            out_v[i, pl.ds(k * LANES, LANES)] = accs[k] + accs[EG + k]

    start(0, rows_a, par_a, sem_a)

    def pair_step(j, _):
        e = pl.multiple_of(j * 2, 2)
        start(e + 1, rows_b, par_b, sem_b)
        wait(rows_a, par_a, sem_a)
        reduce(rows_a, par_a, e)

        @pl.when(j < BPW // 2 - 1)
        def _():
            start(e + 2, rows_a, par_a, sem_a)

        wait(rows_b, par_b, sem_b)
        reduce(rows_b, par_b, e + 1)
        return 0

    lax.fori_loop(0, BPW // 2, pair_step, 0)
    pltpu.sync_copy(out_v, out_hbm.at[pl.ds(base, BPW)])


def _pooled_sum(idx, par, tp):
    mesh = plsc.VectorSubcoreMesh(core_axis_name="c", subcore_axis_name="s")
    f = functools.partial(
        pl.kernel,
        mesh=mesh,
        out_type=jax.ShapeDtypeStruct((BATCH, EMBED), jnp.float32),
        scratch_types=[
            pltpu.VMEM((BPW * SEQ,), jnp.int32),
            pltpu.VMEM((SEQ, PAIR), jnp.float32),
            pltpu.VMEM((SEQ, PAIR), jnp.float32),
            pltpu.VMEM((SEQ + 8,), jnp.int32),
            pltpu.VMEM((SEQ + 8,), jnp.int32),
            pltpu.VMEM((SEQ, EMBED), jnp.float32),
            pltpu.VMEM((BPW, EMBED), jnp.float32),
            pltpu.SemaphoreType.DMA,
            pltpu.SemaphoreType.DMA,
        ],
        compiler_params=pltpu.CompilerParams(needs_layout_passes=False),
    )(_pool_body)
    return f(idx, par, tp)


def _mlp_body(p_ref, w1_ref, b1_ref, w2_ref, b2_ref, w3_ref, b3_ref, o_ref):
    p = p_ref[...] * (1.0 / SEQ)
    h = jnp.dot(p, w1_ref[...], preferred_element_type=jnp.float32)
    h = jnp.maximum(h + b1_ref[...], 0.0)
    h = jnp.dot(h, w2_ref[...], preferred_element_type=jnp.float32)
    h = jnp.maximum(h + b2_ref[...], 0.0)
    o_ref[...] = jnp.dot(h, w3_ref[...],
                         preferred_element_type=jnp.float32) + b3_ref[...]


def _mlp(pooled_sum, W1, b1, W2, b2, W3, b3):
    pad = 128 - CLASSES
    W3p = jnp.pad(W3, ((0, 0), (0, pad)))
    b3p = jnp.pad(b3, (0, pad)).reshape(1, 128)
    out = pl.pallas_call(
        _mlp_body,
        out_shape=jax.ShapeDtypeStruct((BATCH, 128), jnp.float32),
    )(pooled_sum, W1, b1.reshape(1, HIDDEN), W2, b2.reshape(1, HIDDEN),
      W3p, b3p)
    return out[:, :CLASSES]


def kernel(x, table, W1, b1, W2, b2, W3, b3):
    tp = _pair_table(table.T)
    xf = x.astype(jnp.int32).reshape(BATCH * SEQ)
    pooled_sum = _pooled_sum(xf >> 1, (xf & 1) * EMBED, tp)
    return _mlp(pooled_sum, W1, b1, W2, b2, W3, b3)


# speedup vs baseline: 4.6003x; 4.6003x over previous
import functools

import jax
import jax.numpy as jnp
from jax import lax
from jax.experimental import pallas as pl
from jax.experimental.pallas import tpu as pltpu
from jax.experimental.pallas import tpu_sc as plsc

VOCAB = 1000000
EMBED = 64
HIDDEN = 256
CLASSES = 5
BATCH = 4096
SEQ = 200

_INFO = plsc.get_sparse_core_info()
NC = _INFO.num_cores
NS = _INFO.num_subcores
LANES = _INFO.num_lanes
NW = NC * NS
BPW = BATCH // NW
C0 = 104
C1 = SEQ - C0
EG = EMBED // LANES
HALF = SEQ // 2
PAIR = 2 * EMBED
CB = 16384


def _pair_body(tt_ref, o_ref):
    xt = tt_ref[...].T
    o_ref[...] = jnp.concatenate([xt, xt], axis=1)


def _pair_table(tt):
    nb = (VOCAB + CB - 1) // CB
    return pl.pallas_call(
        _pair_body,
        grid=(nb,),
        in_specs=[pl.BlockSpec((EMBED, CB), lambda j: (0, j))],
        out_specs=pl.BlockSpec((CB, PAIR), lambda j: (j, 0)),
        out_shape=jax.ShapeDtypeStruct((VOCAB, PAIR), jnp.float32),
    )(tt)


def _pool_body(idx_hbm, tp_hbm, out_hbm, idx_v, rows_a, rows_b, out_v,
               sem_a, sem_b):
    wid = lax.axis_index("s") * NC + lax.axis_index("c")
    base = pl.multiple_of(wid * BPW, BPW)
    pltpu.sync_copy(idx_hbm.at[pl.ds(base * SEQ, BPW * SEQ)], idx_v)

    def start(e, buf, sem):
        off = pl.multiple_of(e * SEQ, 8)
        pltpu.make_async_copy(
            tp_hbm.at[idx_v.at[pl.ds(off, C0)]],
            buf.at[pl.ds(0, C0)], sem).start()
        pltpu.make_async_copy(
            tp_hbm.at[idx_v.at[pl.ds(off + C0, C1)]],
            buf.at[pl.ds(C0, C1)], sem).start()

    def wait(buf, sem):
        pltpu.make_async_copy(
            tp_hbm.at[idx_v.at[pl.ds(0, C0)]],
            buf.at[pl.ds(0, C0)], sem).wait()
        pltpu.make_async_copy(
            tp_hbm.at[idx_v.at[pl.ds(C0, C1)]],
            buf.at[pl.ds(C0, C1)], sem).wait()

    def reduce(buf, i):
        def body(s, accs):
            new = []
            for j in range(2):
                for k in range(EG):
                    new.append(accs[j * EG + k]
                               + buf[j * HALF + s, pl.ds(k * LANES, LANES)])
            return tuple(new)

        zero = jnp.zeros((LANES,), jnp.float32)
        accs = lax.fori_loop(0, HALF, body, (zero,) * (2 * EG))
        for k in range(EG):
            out_v[i, pl.ds(k * LANES, LANES)] = accs[k] + accs[EG + k]

    start(0, rows_a, sem_a)

    def pair_step(j, _):
        e = pl.multiple_of(j * 2, 2)
        start(e + 1, rows_b, sem_b)
        wait(rows_a, sem_a)
        reduce(rows_a, e)

        @pl.when(j < BPW // 2 - 1)
        def _():
            start(e + 2, rows_a, sem_a)

        wait(rows_b, sem_b)
        reduce(rows_b, e + 1)
        return 0

    lax.fori_loop(0, BPW // 2, pair_step, 0)
    pltpu.sync_copy(out_v, out_hbm.at[pl.ds(base, BPW)])


def _pooled_sum(idx, tp):
    mesh = plsc.VectorSubcoreMesh(core_axis_name="c", subcore_axis_name="s")
    f = functools.partial(
        pl.kernel,
        mesh=mesh,
        out_type=jax.ShapeDtypeStruct((BATCH, EMBED), jnp.float32),
        scratch_types=[
            pltpu.VMEM((BPW * SEQ,), jnp.int32),
            pltpu.VMEM((SEQ, PAIR), jnp.float32),
            pltpu.VMEM((SEQ, PAIR), jnp.float32),
            pltpu.VMEM((BPW, EMBED), jnp.float32),
            pltpu.SemaphoreType.DMA,
            pltpu.SemaphoreType.DMA,
        ],
    )(_pool_body)
    return f(idx, tp)


def _mlp_body(p_ref, w1_ref, b1_ref, w2_ref, b2_ref, w3_ref, b3_ref, o_ref):
    p = p_ref[...] * (1.0 / SEQ)
    h = jnp.dot(p, w1_ref[...], preferred_element_type=jnp.float32)
    h = jnp.maximum(h + b1_ref[...], 0.0)
    h = jnp.dot(h, w2_ref[...], preferred_element_type=jnp.float32)
    h = jnp.maximum(h + b2_ref[...], 0.0)
    o_ref[...] = jnp.dot(h, w3_ref[...],
                         preferred_element_type=jnp.float32) + b3_ref[...]


def _mlp(pooled_sum, W1, b1, W2, b2, W3, b3):
    pad = 128 - CLASSES
    W3p = jnp.pad(W3, ((0, 0), (0, pad)))
    b3p = jnp.pad(b3, (0, pad)).reshape(1, 128)
    out = pl.pallas_call(
        _mlp_body,
        out_shape=jax.ShapeDtypeStruct((BATCH, 128), jnp.float32),
    )(pooled_sum, W1, b1.reshape(1, HIDDEN), W2, b2.reshape(1, HIDDEN),
      W3p, b3p)
    return out[:, :CLASSES]


def kernel(x, table, W1, b1, W2, b2, W3, b3):
    tp = _pair_table(table.T)
    xf = x.astype(jnp.int32).reshape(BATCH * SEQ)
    pooled_sum = _pooled_sum(xf, tp)
    return _mlp(pooled_sum, W1, b1, W2, b2, W3, b3)


# speedup vs baseline: 4.6838x; 1.0182x over previous
import functools

import jax
import jax.numpy as jnp
from jax import lax
from jax.experimental import pallas as pl
from jax.experimental.pallas import tpu as pltpu
from jax.experimental.pallas import tpu_sc as plsc

VOCAB = 1000000
EMBED = 64
HIDDEN = 256
CLASSES = 5
BATCH = 4096
SEQ = 200

_INFO = plsc.get_sparse_core_info()
NC = _INFO.num_cores
NS = _INFO.num_subcores
LANES = _INFO.num_lanes
NW = NC * NS
BPW = BATCH // NW
C0 = 104
C1 = SEQ - C0
EG = EMBED // LANES
HALF = SEQ // 2
PAIR = 2 * EMBED
CB = 20480


def _pair_body(tt_ref, o_ref):
    xt = tt_ref[...].T
    o_ref[...] = jnp.concatenate([xt, xt], axis=1)


def _pair_table(tt):
    nb = (VOCAB + CB - 1) // CB
    return pl.pallas_call(
        _pair_body,
        grid=(nb,),
        in_specs=[pl.BlockSpec((EMBED, CB), lambda j: (0, j))],
        out_specs=pl.BlockSpec((CB, PAIR), lambda j: (j, 0)),
        out_shape=jax.ShapeDtypeStruct((VOCAB, PAIR), jnp.float32),
    )(tt)


def _pool_body(idx_hbm, tp_hbm, out_hbm, idx_v, rows_a, rows_b, out_v,
               sem_a, sem_b):
    wid = lax.axis_index("s") * NC + lax.axis_index("c")
    base = pl.multiple_of(wid * BPW, BPW)
    pltpu.sync_copy(idx_hbm.at[pl.ds(base * SEQ, BPW * SEQ)], idx_v)

    def start(e, buf, sem):
        off = pl.multiple_of(e * SEQ, 8)
        pltpu.make_async_copy(
            tp_hbm.at[idx_v.at[pl.ds(off, C0)]],
            buf.at[pl.ds(0, C0)], sem).start()
        pltpu.make_async_copy(
            tp_hbm.at[idx_v.at[pl.ds(off + C0, C1)]],
            buf.at[pl.ds(C0, C1)], sem).start()

    def wait(buf, sem):
        pltpu.make_async_copy(
            tp_hbm.at[idx_v.at[pl.ds(0, C0)]],
            buf.at[pl.ds(0, C0)], sem).wait()
        pltpu.make_async_copy(
            tp_hbm.at[idx_v.at[pl.ds(C0, C1)]],
            buf.at[pl.ds(C0, C1)], sem).wait()

    def reduce(buf, i):
        def body(s, accs):
            new = []
            for j in range(2):
                for k in range(EG):
                    new.append(accs[j * EG + k]
                               + buf[j * HALF + s, pl.ds(k * LANES, LANES)])
            return tuple(new)

        zero = jnp.zeros((LANES,), jnp.float32)
        accs = lax.fori_loop(0, HALF, body, (zero,) * (2 * EG))
        for k in range(EG):
            out_v[i, pl.ds(k * LANES, LANES)] = accs[k] + accs[EG + k]

    start(0, rows_a, sem_a)

    def pair_step(j, _):
        e = pl.multiple_of(j * 2, 2)
        start(e + 1, rows_b, sem_b)
        wait(rows_a, sem_a)
        reduce(rows_a, e)

        @pl.when(j < BPW // 2 - 1)
        def _():
            start(e + 2, rows_a, sem_a)

        wait(rows_b, sem_b)
        reduce(rows_b, e + 1)
        return 0

    lax.fori_loop(0, BPW // 2, pair_step, 0)
    pltpu.sync_copy(out_v, out_hbm.at[pl.ds(base, BPW)])


def _pooled_sum(idx, tp):
    mesh = plsc.VectorSubcoreMesh(core_axis_name="c", subcore_axis_name="s")
    f = functools.partial(
        pl.kernel,
        mesh=mesh,
        out_type=jax.ShapeDtypeStruct((BATCH, EMBED), jnp.float32),
        scratch_types=[
            pltpu.VMEM((BPW * SEQ,), jnp.int32),
            pltpu.VMEM((SEQ, PAIR), jnp.float32),
            pltpu.VMEM((SEQ, PAIR), jnp.float32),
            pltpu.VMEM((BPW, EMBED), jnp.float32),
            pltpu.SemaphoreType.DMA,
            pltpu.SemaphoreType.DMA,
        ],
    )(_pool_body)
    return f(idx, tp)


def _mlp_body(p_ref, w1_ref, b1_ref, w2_ref, b2_ref, w3_ref, b3_ref, o_ref):
    p = p_ref[...] * (1.0 / SEQ)
    h = jnp.dot(p, w1_ref[...], preferred_element_type=jnp.float32)
    h = jnp.maximum(h + b1_ref[...], 0.0)
    h = jnp.dot(h, w2_ref[...], preferred_element_type=jnp.float32)
    h = jnp.maximum(h + b2_ref[...], 0.0)
    o_ref[...] = jnp.dot(h, w3_ref[...],
                         preferred_element_type=jnp.float32) + b3_ref[...]


def _mlp(pooled_sum, W1, b1, W2, b2, W3, b3):
    pad = 128 - CLASSES
    W3p = jnp.pad(W3, ((0, 0), (0, pad)))
    b3p = jnp.pad(b3, (0, pad)).reshape(1, 128)
    out = pl.pallas_call(
        _mlp_body,
        out_shape=jax.ShapeDtypeStruct((BATCH, 128), jnp.float32),
    )(pooled_sum, W1, b1.reshape(1, HIDDEN), W2, b2.reshape(1, HIDDEN),
      W3p, b3p)
    return out[:, :CLASSES]


def kernel(x, table, W1, b1, W2, b2, W3, b3):
    tp = _pair_table(table.T)
    xf = x.astype(jnp.int32).reshape(BATCH * SEQ)
    pooled_sum = _pooled_sum(xf, tp)
    return _mlp(pooled_sum, W1, b1, W2, b2, W3, b3)


# speedup vs baseline: 4.7141x; 1.0065x over previous
import functools

import jax
import jax.numpy as jnp
from jax import lax
from jax.experimental import pallas as pl
from jax.experimental.pallas import tpu as pltpu
from jax.experimental.pallas import tpu_sc as plsc

VOCAB = 1000000
EMBED = 64
HIDDEN = 256
CLASSES = 5
BATCH = 4096
SEQ = 200

_INFO = plsc.get_sparse_core_info()
NC = _INFO.num_cores
NS = _INFO.num_subcores
LANES = _INFO.num_lanes
NW = NC * NS
BPW = BATCH // NW
C0 = 104
C1 = SEQ - C0
EG = EMBED // LANES
HALF = SEQ // 2
PAIR = 2 * EMBED
CB = 24576


def _pair_body(tt_ref, o_ref):
    xt = tt_ref[...].T
    o_ref[...] = jnp.concatenate([xt, xt], axis=1)


def _pair_table(tt):
    nb = (VOCAB + CB - 1) // CB
    return pl.pallas_call(
        _pair_body,
        grid=(nb,),
        in_specs=[pl.BlockSpec((EMBED, CB), lambda j: (0, j))],
        out_specs=pl.BlockSpec((CB, PAIR), lambda j: (j, 0)),
        out_shape=jax.ShapeDtypeStruct((VOCAB, PAIR), jnp.float32),
    )(tt)


def _pool_body(idx_hbm, tp_hbm, out_hbm, idx_v, rows_a, rows_b, out_v,
               sem_a, sem_b):
    wid = lax.axis_index("s") * NC + lax.axis_index("c")
    base = pl.multiple_of(wid * BPW, BPW)
    pltpu.sync_copy(idx_hbm.at[pl.ds(base * SEQ, BPW * SEQ)], idx_v)

    def start(e, buf, sem):
        off = pl.multiple_of(e * SEQ, 8)
        pltpu.make_async_copy(
            tp_hbm.at[idx_v.at[pl.ds(off, C0)]],
            buf.at[pl.ds(0, C0)], sem).start()
        pltpu.make_async_copy(
            tp_hbm.at[idx_v.at[pl.ds(off + C0, C1)]],
            buf.at[pl.ds(C0, C1)], sem).start()

    def wait(buf, sem):
        pltpu.make_async_copy(
            tp_hbm.at[idx_v.at[pl.ds(0, C0)]],
            buf.at[pl.ds(0, C0)], sem).wait()
        pltpu.make_async_copy(
            tp_hbm.at[idx_v.at[pl.ds(C0, C1)]],
            buf.at[pl.ds(C0, C1)], sem).wait()

    def reduce(buf, i):
        def body(s, accs):
            new = []
            for j in range(2):
                for k in range(EG):
                    new.append(accs[j * EG + k]
                               + buf[j * HALF + s, pl.ds(k * LANES, LANES)])
            return tuple(new)

        zero = jnp.zeros((LANES,), jnp.float32)
        accs = lax.fori_loop(0, HALF, body, (zero,) * (2 * EG))
        for k in range(EG):
            out_v[i, pl.ds(k * LANES, LANES)] = accs[k] + accs[EG + k]

    start(0, rows_a, sem_a)

    def pair_step(j, _):
        e = pl.multiple_of(j * 2, 2)
        start(e + 1, rows_b, sem_b)
        wait(rows_a, sem_a)
        reduce(rows_a, e)

        @pl.when(j < BPW // 2 - 1)
        def _():
            start(e + 2, rows_a, sem_a)

        wait(rows_b, sem_b)
        reduce(rows_b, e + 1)
        return 0

    lax.fori_loop(0, BPW // 2, pair_step, 0)
    pltpu.sync_copy(out_v, out_hbm.at[pl.ds(base, BPW)])


def _pooled_sum(idx, tp):
    mesh = plsc.VectorSubcoreMesh(core_axis_name="c", subcore_axis_name="s")
    f = functools.partial(
        pl.kernel,
        mesh=mesh,
        out_type=jax.ShapeDtypeStruct((BATCH, EMBED), jnp.float32),
        scratch_types=[
            pltpu.VMEM((BPW * SEQ,), jnp.int32),
            pltpu.VMEM((SEQ, PAIR), jnp.float32),
            pltpu.VMEM((SEQ, PAIR), jnp.float32),
            pltpu.VMEM((BPW, EMBED), jnp.float32),
            pltpu.SemaphoreType.DMA,
            pltpu.SemaphoreType.DMA,
        ],
    )(_pool_body)
    return f(idx, tp)


def _mlp_body(p_ref, w1_ref, b1_ref, w2_ref, b2_ref, w3_ref, b3_ref, o_ref):
    p = p_ref[...] * (1.0 / SEQ)
    h = jnp.dot(p, w1_ref[...], preferred_element_type=jnp.float32)
    h = jnp.maximum(h + b1_ref[...], 0.0)
    h = jnp.dot(h, w2_ref[...], preferred_element_type=jnp.float32)
    h = jnp.maximum(h + b2_ref[...], 0.0)
    o_ref[...] = jnp.dot(h, w3_ref[...],
                         preferred_element_type=jnp.float32) + b3_ref[...]


def _mlp(pooled_sum, W1, b1, W2, b2, W3, b3):
    pad = 128 - CLASSES
    W3p = jnp.pad(W3, ((0, 0), (0, pad)))
    b3p = jnp.pad(b3, (0, pad)).reshape(1, 128)
    out = pl.pallas_call(
        _mlp_body,
        out_shape=jax.ShapeDtypeStruct((BATCH, 128), jnp.float32),
    )(pooled_sum, W1, b1.reshape(1, HIDDEN), W2, b2.reshape(1, HIDDEN),
      W3p, b3p)
    return out[:, :CLASSES]


def kernel(x, table, W1, b1, W2, b2, W3, b3):
    tp = _pair_table(table.T)
    xf = x.astype(jnp.int32).reshape(BATCH * SEQ)
    pooled_sum = _pooled_sum(xf, tp)
    return _mlp(pooled_sum, W1, b1, W2, b2, W3, b3)


# speedup vs baseline: 4.9321x; 1.0463x over previous
import functools

import jax
import jax.numpy as jnp
from jax import lax
from jax.experimental import pallas as pl
from jax.experimental.pallas import tpu as pltpu
from jax.experimental.pallas import tpu_sc as plsc

VOCAB = 1000000
EMBED = 64
HIDDEN = 256
CLASSES = 5
BATCH = 4096
SEQ = 200

_INFO = plsc.get_sparse_core_info()
NC = _INFO.num_cores
NS = _INFO.num_subcores
LANES = _INFO.num_lanes
NW = NC * NS
BPW = BATCH // NW
C0 = 104
C1 = SEQ - C0
EG = EMBED // LANES
HALF = SEQ // 2
PAIR = 2 * EMBED
CB = 24576


def _pair_body(tt_ref, o_ref):
    xt = tt_ref[...].T
    o_ref[...] = jnp.concatenate([xt, xt], axis=1)


def _pair_table(tt):
    nb = (VOCAB + CB - 1) // CB
    return pl.pallas_call(
        _pair_body,
        grid=(nb,),
        in_specs=[pl.BlockSpec((EMBED, CB), lambda j: (0, j))],
        out_specs=pl.BlockSpec((CB, PAIR), lambda j: (j, 0)),
        out_shape=jax.ShapeDtypeStruct((VOCAB, PAIR), jnp.float32),
    )(tt)


def _pool_body(idx_hbm, tp_hbm, out_hbm, idx_v, rows_a, rows_b, rows_c,
               out_v, sem_a, sem_b, sem_c):
    wid = lax.axis_index("s") * NC + lax.axis_index("c")
    base = pl.multiple_of(wid * BPW, BPW)
    pltpu.sync_copy(idx_hbm.at[pl.ds(base * SEQ, BPW * SEQ)], idx_v)

    def start(e, buf, sem):
        off = pl.multiple_of(e * SEQ, 8)
        pltpu.make_async_copy(
            tp_hbm.at[idx_v.at[pl.ds(off, C0)]],
            buf.at[pl.ds(0, C0)], sem).start()
        pltpu.make_async_copy(
            tp_hbm.at[idx_v.at[pl.ds(off + C0, C1)]],
            buf.at[pl.ds(C0, C1)], sem).start()

    def wait(buf, sem):
        pltpu.make_async_copy(
            tp_hbm.at[idx_v.at[pl.ds(0, C0)]],
            buf.at[pl.ds(0, C0)], sem).wait()
        pltpu.make_async_copy(
            tp_hbm.at[idx_v.at[pl.ds(C0, C1)]],
            buf.at[pl.ds(C0, C1)], sem).wait()

    def reduce(buf, i):
        def body(s, accs):
            new = []
            for j in range(2):
                for k in range(EG):
                    new.append(accs[j * EG + k]
                               + buf[j * HALF + s, pl.ds(k * LANES, LANES)])
            return tuple(new)

        zero = jnp.zeros((LANES,), jnp.float32)
        accs = lax.fori_loop(0, HALF, body, (zero,) * (2 * EG))
        for k in range(EG):
            out_v[i, pl.ds(k * LANES, LANES)] = accs[k] + accs[EG + k]

    start(0, rows_a, sem_a)
    start(1, rows_b, sem_b)

    def tri_step(j, _):
        e = pl.multiple_of(j * 3, 3)

        @pl.when(e + 2 < BPW)
        def _():
            start(e + 2, rows_c, sem_c)

        wait(rows_a, sem_a)
        reduce(rows_a, e)

        @pl.when(e + 3 < BPW)
        def _():
            start(e + 3, rows_a, sem_a)

        wait(rows_b, sem_b)
        reduce(rows_b, e + 1)

        @pl.when(e + 4 < BPW)
        def _():
            start(e + 4, rows_b, sem_b)

        @pl.when(e + 2 < BPW)
        def _():
            wait(rows_c, sem_c)
            reduce(rows_c, e + 2)

        return 0

    lax.fori_loop(0, (BPW + 2) // 3, tri_step, 0)
    pltpu.sync_copy(out_v, out_hbm.at[pl.ds(base, BPW)])


def _pooled_sum(idx, tp):
    mesh = plsc.VectorSubcoreMesh(core_axis_name="c", subcore_axis_name="s")
    f = functools.partial(
        pl.kernel,
        mesh=mesh,
        out_type=jax.ShapeDtypeStruct((BATCH, EMBED), jnp.float32),
        scratch_types=[
            pltpu.VMEM((BPW * SEQ,), jnp.int32),
            pltpu.VMEM((SEQ, PAIR), jnp.float32),
            pltpu.VMEM((SEQ, PAIR), jnp.float32),
            pltpu.VMEM((SEQ, PAIR), jnp.float32),
            pltpu.VMEM((BPW, EMBED), jnp.float32),
            pltpu.SemaphoreType.DMA,
            pltpu.SemaphoreType.DMA,
            pltpu.SemaphoreType.DMA,
        ],
    )(_pool_body)
    return f(idx, tp)


def _mlp_body(p_ref, w1_ref, b1_ref, w2_ref, b2_ref, w3_ref, b3_ref, o_ref):
    p = p_ref[...] * (1.0 / SEQ)
    h = jnp.dot(p, w1_ref[...], preferred_element_type=jnp.float32)
    h = jnp.maximum(h + b1_ref[...], 0.0)
    h = jnp.dot(h, w2_ref[...], preferred_element_type=jnp.float32)
    h = jnp.maximum(h + b2_ref[...], 0.0)
    o_ref[...] = jnp.dot(h, w3_ref[...],
                         preferred_element_type=jnp.float32) + b3_ref[...]


def _mlp(pooled_sum, W1, b1, W2, b2, W3, b3):
    pad = 128 - CLASSES
    W3p = jnp.pad(W3, ((0, 0), (0, pad)))
    b3p = jnp.pad(b3, (0, pad)).reshape(1, 128)
    out = pl.pallas_call(
        _mlp_body,
        out_shape=jax.ShapeDtypeStruct((BATCH, 128), jnp.float32),
    )(pooled_sum, W1, b1.reshape(1, HIDDEN), W2, b2.reshape(1, HIDDEN),
      W3p, b3p)
    return out[:, :CLASSES]


def kernel(x, table, W1, b1, W2, b2, W3, b3):
    tp = _pair_table(table.T)
    xf = x.astype(jnp.int32).reshape(BATCH * SEQ)
    pooled_sum = _pooled_sum(xf, tp)
    return _mlp(pooled_sum, W1, b1, W2, b2, W3, b3)


# speedup vs baseline: 4.9394x; 1.0015x over previous
import functools

import jax
import jax.numpy as jnp
from jax import lax
from jax.experimental import pallas as pl
from jax.experimental.pallas import tpu as pltpu
from jax.experimental.pallas import tpu_sc as plsc

VOCAB = 1000000
EMBED = 64
HIDDEN = 256
CLASSES = 5
BATCH = 4096
SEQ = 200

_INFO = plsc.get_sparse_core_info()
NC = _INFO.num_cores
NS = _INFO.num_subcores
LANES = _INFO.num_lanes
NW = NC * NS
BPW = BATCH // NW
C0 = 104
C1 = SEQ - C0
EG = EMBED // LANES
HALF = SEQ // 2
PAIR = 2 * EMBED
CB = 28672


def _pair_body(tt_ref, o_ref):
    xt = tt_ref[...].T
    o_ref[...] = jnp.concatenate([xt, xt], axis=1)


def _pair_table(tt):
    nb = (VOCAB + CB - 1) // CB
    return pl.pallas_call(
        _pair_body,
        grid=(nb,),
        in_specs=[pl.BlockSpec((EMBED, CB), lambda j: (0, j))],
        out_specs=pl.BlockSpec((CB, PAIR), lambda j: (j, 0)),
        out_shape=jax.ShapeDtypeStruct((VOCAB, PAIR), jnp.float32),
    )(tt)


def _pool_body(idx_hbm, tp_hbm, out_hbm, idx_v, rows_a, rows_b, rows_c,
               out_v, sem_a, sem_b, sem_c):
    wid = lax.axis_index("s") * NC + lax.axis_index("c")
    base = pl.multiple_of(wid * BPW, BPW)
    pltpu.sync_copy(idx_hbm.at[pl.ds(base * SEQ, BPW * SEQ)], idx_v)

    def start(e, buf, sem):
        off = pl.multiple_of(e * SEQ, 8)
        pltpu.make_async_copy(
            tp_hbm.at[idx_v.at[pl.ds(off, C0)]],
            buf.at[pl.ds(0, C0)], sem).start()
        pltpu.make_async_copy(
            tp_hbm.at[idx_v.at[pl.ds(off + C0, C1)]],
            buf.at[pl.ds(C0, C1)], sem).start()

    def wait(buf, sem):
        pltpu.make_async_copy(
            tp_hbm.at[idx_v.at[pl.ds(0, C0)]],
            buf.at[pl.ds(0, C0)], sem).wait()
        pltpu.make_async_copy(
            tp_hbm.at[idx_v.at[pl.ds(C0, C1)]],
            buf.at[pl.ds(C0, C1)], sem).wait()

    def reduce(buf, i):
        def body(s, accs):
            new = []
            for j in range(2):
                for k in range(EG):
                    new.append(accs[j * EG + k]
                               + buf[j * HALF + s, pl.ds(k * LANES, LANES)])
            return tuple(new)

        zero = jnp.zeros((LANES,), jnp.float32)
        accs = lax.fori_loop(0, HALF, body, (zero,) * (2 * EG))
        for k in range(EG):
            out_v[i, pl.ds(k * LANES, LANES)] = accs[k] + accs[EG + k]

    start(0, rows_a, sem_a)
    start(1, rows_b, sem_b)

    def tri_step(j, _):
        e = pl.multiple_of(j * 3, 3)

        @pl.when(e + 2 < BPW)
        def _():
            start(e + 2, rows_c, sem_c)

        wait(rows_a, sem_a)
        reduce(rows_a, e)

        @pl.when(e + 3 < BPW)
        def _():
            start(e + 3, rows_a, sem_a)

        wait(rows_b, sem_b)
        reduce(rows_b, e + 1)

        @pl.when(e + 4 < BPW)
        def _():
            start(e + 4, rows_b, sem_b)

        @pl.when(e + 2 < BPW)
        def _():
            wait(rows_c, sem_c)
            reduce(rows_c, e + 2)

        return 0

    lax.fori_loop(0, (BPW + 2) // 3, tri_step, 0)
    pltpu.sync_copy(out_v, out_hbm.at[pl.ds(base, BPW)])


def _pooled_sum(idx, tp):
    mesh = plsc.VectorSubcoreMesh(core_axis_name="c", subcore_axis_name="s")
    f = functools.partial(
        pl.kernel,
        mesh=mesh,
        out_type=jax.ShapeDtypeStruct((BATCH, EMBED), jnp.float32),
        scratch_types=[
            pltpu.VMEM((BPW * SEQ,), jnp.int32),
            pltpu.VMEM((SEQ, PAIR), jnp.float32),
            pltpu.VMEM((SEQ, PAIR), jnp.float32),
            pltpu.VMEM((SEQ, PAIR), jnp.float32),
            pltpu.VMEM((BPW, EMBED), jnp.float32),
            pltpu.SemaphoreType.DMA,
            pltpu.SemaphoreType.DMA,
            pltpu.SemaphoreType.DMA,
        ],
    )(_pool_body)
    return f(idx, tp)


def _mlp_body(p_ref, w1_ref, b1_ref, w2_ref, b2_ref, w3_ref, b3_ref, o_ref):
    p = p_ref[...] * (1.0 / SEQ)
    h = jnp.dot(p, w1_ref[...], preferred_element_type=jnp.float32)
    h = jnp.maximum(h + b1_ref[...], 0.0)
    h = jnp.dot(h, w2_ref[...], preferred_element_type=jnp.float32)
    h = jnp.maximum(h + b2_ref[...], 0.0)
    o_ref[...] = jnp.dot(h, w3_ref[...],
                         preferred_element_type=jnp.float32) + b3_ref[...]


def _mlp(pooled_sum, W1, b1, W2, b2, W3, b3):
    pad = 128 - CLASSES
    W3p = jnp.pad(W3, ((0, 0), (0, pad)))
    b3p = jnp.pad(b3, (0, pad)).reshape(1, 128)
    out = pl.pallas_call(
        _mlp_body,
        out_shape=jax.ShapeDtypeStruct((BATCH, 128), jnp.float32),
    )(pooled_sum, W1, b1.reshape(1, HIDDEN), W2, b2.reshape(1, HIDDEN),
      W3p, b3p)
    return out[:, :CLASSES]


def kernel(x, table, W1, b1, W2, b2, W3, b3):
    tp = _pair_table(table.T)
    xf = x.astype(jnp.int32).reshape(BATCH * SEQ)
    pooled_sum = _pooled_sum(xf, tp)
    return _mlp(pooled_sum, W1, b1, W2, b2, W3, b3)
